# baseline (speedup 1.0000x reference)
import functools

import jax
import jax.numpy as jnp
from jax import lax
from jax.experimental import pallas as pl
from jax.experimental.pallas import tpu as pltpu
from jax.experimental.pallas import tpu_sc as plsc

_NSUB = 16
_NCORE = 2


def _round_up(v, m):
    return ((v + m - 1) // m) * m


def _sc_degree(dst_raw2, npad):
    rows_n = dst_raw2.shape[0]
    nw = _NSUB * _NCORE
    rt = _round_up(-(-rows_n // nw), 8)
    npt = npad // _NSUB
    mesh = plsc.VectorSubcoreMesh(core_axis_name="c", subcore_axis_name="s")

    @functools.partial(
        pl.kernel,
        out_type=(jax.ShapeDtypeStruct((npad,), jnp.float32),
                  jax.ShapeDtypeStruct((npad,), jnp.float32)),
        mesh=mesh,
        scratch_types=[
            pltpu.VMEM_SHARED((npad,), jnp.float32),
            pltpu.VMEM((rt, _EW), jnp.int32),
            pltpu.VMEM((npt,), jnp.float32),
        ],
    )
    def k(dst_h, deg0_h, deg1_h, deg_sh, dst_v, ones_v):
        c = lax.axis_index("c")
        s = lax.axis_index("s")

        def fill(i, carry):
            ones_v[pl.ds(i * 16, 16)] = jnp.ones((16,), jnp.float32)
            return carry

        lax.fori_loop(0, npt // 16, fill, 0)
        pltpu.sync_copy(ones_v, deg_sh.at[pl.ds(s * npt, npt)])
        wid = c * _NSUB + s
        base = wid * rt
        base_ld = pl.multiple_of(jnp.minimum(base, rows_n - rt), 8)
        off = base - base_ld
        cnt = jnp.clip(rows_n - base, 0, rt)
        pltpu.sync_copy(dst_h.at[pl.ds(base_ld, rt)], dst_v)
        plsc.subcore_barrier()

        def body(j, carry):
            pltpu.sync_copy(ones_v.at[pl.ds(0, _EW)],
                            deg_sh.at[dst_v.at[off + j]], add=True)
            return carry

        lax.fori_loop(0, cnt, body, 0)
        plsc.subcore_barrier()

        @pl.when(c == 0)
        def _():
            pltpu.sync_copy(deg_sh.at[pl.ds(s * npt, npt)],
                            deg0_h.at[pl.ds(s * npt, npt)])

        @pl.when(c == 1)
        def _():
            pltpu.sync_copy(deg_sh.at[pl.ds(s * npt, npt)],
                            deg1_h.at[pl.ds(s * npt, npt)])

    return k(dst_raw2)


_EW = 64
_CH = 32
_NBUF = 4
_LA = 2


def _edge_pipeline(gather_start, gather_wait, acc_sh, src_h, dst_h,
                   src_v, dst_v, rows, gsems, ssems, row0, nrows):
    nblk = _CH // _NBUF

    def sc_desc(p, jj):
        return pltpu.make_async_copy(rows[p], acc_sh.at[dst_v.at[jj]],
                                     ssems[p])

    def chunk(jc, carry):
        base = row0 + jc * _CH
        pltpu.sync_copy(src_h.at[pl.ds(base, _CH)], src_v)
        pltpu.sync_copy(dst_h.at[pl.ds(base, _CH)], dst_v)
        for r in range(_LA):
            pl.when(jc > 0)(lambda r=r: sc_desc(r % _NBUF, r).wait())
            gather_start(src_v.at[r], rows[r % _NBUF], gsems[r % _NBUF])

        def blk(jb, carry2):
            for u in range(_NBUF):
                r = jb * _NBUF + u
                gather_wait(src_v.at[r], rows[u], gsems[u])
                sc_desc(u, r).start(add=True)
                rn = r + _LA
                pn = (u + _LA) % _NBUF
                if u + _LA < _NBUF:
                    def pre(rn=rn, pn=pn):
                        gather_start(src_v.at[rn], rows[pn], gsems[pn])
                    pl.when(jnp.logical_or(jc > 0, jb > 0))(
                        lambda rn=rn, pn=pn: sc_desc(pn, rn).wait())
                    pre()
                else:
                    def pre(rn=rn, pn=pn):
                        sc_desc(pn, rn).wait()
                        gather_start(src_v.at[rn], rows[pn], gsems[pn])
                    pl.when(jb < nblk - 1)(pre)
            return carry2

        return lax.fori_loop(0, nblk, blk, carry)

    lax.fori_loop(0, nrows // _CH, chunk, 0)
    for p in range(_NBUF):
        sc_desc(p, p).wait()


def _sc_scatter(hp0, hp1, src2, dst2, tsrc, tdst, npad, dc, rows_per_tec):
    npt = npad // _NSUB
    mesh = plsc.VectorSubcoreMesh(core_axis_name="c", subcore_axis_name="s")

    @functools.partial(
        pl.kernel,
        out_type=(jax.ShapeDtypeStruct((npad, dc), jnp.float32),
                  jax.ShapeDtypeStruct((npad, dc), jnp.float32)),
        mesh=mesh,
        scratch_types=[
            pltpu.VMEM_SHARED((npad, dc), jnp.float32),
            pltpu.VMEM((_CH, _EW), jnp.int32),
            pltpu.VMEM((_CH, _EW), jnp.int32),
        ] + [pltpu.VMEM((_EW, dc), jnp.float32)] * _NBUF
          + [pltpu.SemaphoreType.DMA] * (2 * _NBUF),
    )
    def k(hp0_h, hp1_h, src_h, dst_h, tsrc_h, tdst_h, acc0_h, acc1_h,
          acc_sh, src_v, dst_v, *bufs):
        rows = bufs[:_NBUF]
        gsems = bufs[_NBUF:2 * _NBUF]
        ssems = bufs[2 * _NBUF:]
        c = lax.axis_index("c")
        s = lax.axis_index("s")

        @pl.when(c == 0)
        def _():
            pltpu.sync_copy(hp0_h.at[pl.ds(s * npt, npt)],
                            acc_sh.at[pl.ds(s * npt, npt)])

        @pl.when(c == 1)
        def _():
            pltpu.sync_copy(hp1_h.at[pl.ds(s * npt, npt)],
                            acc_sh.at[pl.ds(s * npt, npt)])

        plsc.subcore_barrier()

        def gather_start(idx_ref, buf, gsem):
            @pl.when(c == 0)
            def _():
                pltpu.make_async_copy(hp0_h.at[idx_ref], buf, gsem).start()

            @pl.when(c == 1)
            def _():
                pltpu.make_async_copy(hp1_h.at[idx_ref], buf, gsem).start()

        def gather_wait(idx_ref, buf, gsem):
            pltpu.make_async_copy(hp0_h.at[idx_ref], buf, gsem).wait()

        @pl.when(s < _NSUB - 1)
        def _():
            _edge_pipeline(gather_start, gather_wait, acc_sh, src_h, dst_h,
                           src_v, dst_v, rows, gsems, ssems,
                           s * rows_per_tec, rows_per_tec)

        @pl.when(s == _NSUB - 1)
        def _():
            _edge_pipeline(gather_start, gather_wait, acc_sh, tsrc_h, tdst_h,
                           src_v, dst_v, rows, gsems, ssems, 0, rows_per_tec)

        plsc.subcore_barrier()

        @pl.when(c == 0)
        def _():
            pltpu.sync_copy(acc_sh.at[pl.ds(s * npt, npt)],
                            acc0_h.at[pl.ds(s * npt, npt)])

        @pl.when(c == 1)
        def _():
            pltpu.sync_copy(acc_sh.at[pl.ds(s * npt, npt)],
                            acc1_h.at[pl.ds(s * npt, npt)])

    return k(hp0, hp1, src2, dst2, tsrc, tdst)


def _sc_scatter_edge(hp, src2, dst2, tsrc, tdst, npad, d, rows_per_tec):
    npt = npad // _NSUB
    mesh = plsc.VectorSubcoreMesh(core_axis_name="c", subcore_axis_name="s")

    @functools.partial(
        pl.kernel,
        out_type=(jax.ShapeDtypeStruct((npad, d), jnp.float32),
                  jax.ShapeDtypeStruct((npad, d), jnp.float32)),
        mesh=mesh,
        scratch_types=[
            pltpu.VMEM_SHARED((npad, d), jnp.float32),
            pltpu.VMEM((_CH, _EW), jnp.int32),
            pltpu.VMEM((_CH, _EW), jnp.int32),
        ] + [pltpu.VMEM((_EW, d), jnp.float32)] * _NBUF
          + [pltpu.SemaphoreType.DMA] * (2 * _NBUF),
    )
    def k(hp_h, src_h, dst_h, tsrc_h, tdst_h, acca_h, accb_h,
          acc_sh, src_v, dst_v, *bufs):
        rows = bufs[:_NBUF]
        gsems = bufs[_NBUF:2 * _NBUF]
        ssems = bufs[2 * _NBUF:]
        c = lax.axis_index("c")
        s = lax.axis_index("s")

        pltpu.sync_copy(hp_h.at[pl.ds(s * npt, npt)],
                        acc_sh.at[pl.ds(s * npt, npt)])

        plsc.subcore_barrier()

        def gather_start(idx_ref, buf, gsem):
            pltpu.make_async_copy(hp_h.at[idx_ref], buf, gsem).start()

        def gather_wait(idx_ref, buf, gsem):
            pltpu.make_async_copy(hp_h.at[idx_ref], buf, gsem).wait()

        wid = c * _NSUB + s
        nw = _NCORE * _NSUB

        @pl.when(wid < nw - 1)
        def _():
            _edge_pipeline(gather_start, gather_wait, acc_sh, src_h, dst_h,
                           src_v, dst_v, rows, gsems, ssems,
                           wid * rows_per_tec, rows_per_tec)

        @pl.when(wid == nw - 1)
        def _():
            _edge_pipeline(gather_start, gather_wait, acc_sh, tsrc_h, tdst_h,
                           src_v, dst_v, rows, gsems, ssems,
                           rows_per_tec, rows_per_tec)
        plsc.subcore_barrier()

        @pl.when(c == 0)
        def _():
            pltpu.sync_copy(acc_sh.at[pl.ds(s * npt, npt)],
                            acca_h.at[pl.ds(s * npt, npt)])

        @pl.when(c == 1)
        def _():
            pltpu.sync_copy(acc_sh.at[pl.ds(s * npt, npt)],
                            accb_h.at[pl.ds(s * npt, npt)])

    return k(hp, src2, dst2, tsrc, tdst)


def _tc_matmul_scale(x, w, deg0, deg1, npad, bm):
    n, d_in = x.shape
    d_out = w.shape[1]
    dh = d_out // 2

    def body(x_ref, w_ref, d0_ref, d1_ref, o0_ref, o1_ref):
        dis = lax.rsqrt(d0_ref[...] + d1_ref[...] - 1.0)
        h = jnp.dot(x_ref[...], w_ref[...], preferred_element_type=jnp.float32)
        hp = h * dis
        o0_ref[...] = hp[:, :dh]
        o1_ref[...] = hp[:, dh:]

    return pl.pallas_call(
        body,
        grid=(npad // bm,),
        in_specs=[
            pl.BlockSpec((bm, d_in), lambda i: (i, 0)),
            pl.BlockSpec((d_in, d_out), lambda i: (0, 0)),
            pl.BlockSpec((bm, 1), lambda i: (i, 0)),
            pl.BlockSpec((bm, 1), lambda i: (i, 0)),
        ],
        out_specs=[
            pl.BlockSpec((bm, dh), lambda i: (i, 0)),
            pl.BlockSpec((bm, dh), lambda i: (i, 0)),
        ],
        out_shape=[
            jax.ShapeDtypeStruct((npad, dh), jnp.float32),
            jax.ShapeDtypeStruct((npad, dh), jnp.float32),
        ],
    )(x, w, deg0, deg1)


def _tc_relu_matmul_scale(acc0, acc1, deg0, deg1, b, w, npad, bm):
    dc = acc0.shape[1]
    d_out = w.shape[1]

    def body(a0_ref, a1_ref, d0_ref, d1_ref, b_ref, w_ref, o_ref):
        dis = lax.rsqrt(d0_ref[...] + d1_ref[...] - 1.0)
        a = jnp.concatenate([a0_ref[...], a1_ref[...]], axis=1)
        out1 = jnp.maximum(a * dis + b_ref[...], 0.0)
        h = jnp.dot(out1, w_ref[...], preferred_element_type=jnp.float32)
        o_ref[...] = h * dis

    return pl.pallas_call(
        body,
        grid=(npad // bm,),
        in_specs=[
            pl.BlockSpec((bm, dc), lambda i: (i, 0)),
            pl.BlockSpec((bm, dc), lambda i: (i, 0)),
            pl.BlockSpec((bm, 1), lambda i: (i, 0)),
            pl.BlockSpec((bm, 1), lambda i: (i, 0)),
            pl.BlockSpec((1, 2 * dc), lambda i: (0, 0)),
            pl.BlockSpec((2 * dc, d_out), lambda i: (0, 0)),
        ],
        out_specs=pl.BlockSpec((bm, d_out), lambda i: (i, 0)),
        out_shape=jax.ShapeDtypeStruct((npad, d_out), jnp.float32),
    )(acc0, acc1, deg0, deg1, b, w)


def _tc_relu_sum(acca, accb, hp, deg0, deg1, b, nout, npad, bm):
    d = acca.shape[1]

    def body(aa_ref, ab_ref, hp_ref, d0_ref, d1_ref, b_ref, o_ref):
        dis = lax.rsqrt(d0_ref[...] + d1_ref[...] - 1.0)
        a = aa_ref[...] + ab_ref[...] - hp_ref[...]
        o_ref[...] = jnp.maximum(a * dis + b_ref[...], 0.0)

    return pl.pallas_call(
        body,
        grid=(npad // bm,),
        in_specs=[
            pl.BlockSpec((bm, d), lambda i: (i, 0)),
            pl.BlockSpec((bm, d), lambda i: (i, 0)),
            pl.BlockSpec((bm, d), lambda i: (i, 0)),
            pl.BlockSpec((bm, 1), lambda i: (i, 0)),
            pl.BlockSpec((bm, 1), lambda i: (i, 0)),
            pl.BlockSpec((1, d), lambda i: (0, 0)),
        ],
        out_specs=pl.BlockSpec((bm, d), lambda i: (i, 0)),
        out_shape=jax.ShapeDtypeStruct((nout, d), jnp.float32),
    )(acca, accb, hp, deg0, deg1, b)


def kernel(x, edge_index, idx, pre_z1, pre_z2, W1, b1, W2, b2):
    n = x.shape[0]
    e = edge_index.shape[1]
    d_hid = W1.shape[1]
    d_out = W2.shape[1]

    npt = _round_up(-(-n // _NSUB), 64)
    npad = npt * _NSUB
    rows_per_tec = _round_up(-(-e // (_NSUB * _EW)), 2 * _CH)
    epad = rows_per_tec * _NSUB * _EW

    src = edge_index[0]
    dst = edge_index[1]
    pad = epad - e
    pad_ar = jnp.arange(pad, dtype=jnp.int32)
    pad_src = pad_ar % n
    pad_dst = n + pad_ar % (npad - n)
    rows_raw = e // _EW
    tail_lo = (_NSUB - 1) * rows_per_tec
    if (e % _EW == 0 and rows_raw >= tail_lo
            and rows_raw >= (2 * _NSUB - 1) * (rows_per_tec // 2)):
        src2 = src.reshape(rows_raw, _EW)
        dst2 = dst.reshape(rows_raw, _EW)
        tsrc = jnp.concatenate(
            [src[tail_lo * _EW:], pad_src]).reshape(rows_per_tec, _EW)
        tdst = jnp.concatenate(
            [dst[tail_lo * _EW:], pad_dst]).reshape(rows_per_tec, _EW)
    else:
        src2 = jnp.concatenate([src, pad_src]).reshape(epad // _EW, _EW)
        dst2 = jnp.concatenate([dst, pad_dst]).reshape(epad // _EW, _EW)
        tsrc = src2[tail_lo:]
        tdst = dst2[tail_lo:]

    rows_raw = e // _EW
    rt_deg = _round_up(-(-rows_raw // (_NSUB * _NCORE)), 8)
    if e % _EW == 0 and (rows_raw - rt_deg) % 8 == 0 and rows_raw >= rt_deg:
        deg0, deg1 = _sc_degree(dst.reshape(rows_raw, _EW), npad)
    else:
        deg0, deg1 = _sc_degree(dst2, npad)
    deg0r = deg0.reshape(npad, 1)
    deg1r = deg1.reshape(npad, 1)

    bm = npad // 10

    hp0, hp1 = _tc_matmul_scale(x, W1, deg0r, deg1r, npad, bm)
    acc0, acc1 = _sc_scatter(hp0, hp1, src2, dst2, tsrc, tdst, npad,
                             d_hid // 2, rows_per_tec)
    h2p = _tc_relu_matmul_scale(acc0, acc1, deg0r, deg1r,
                                b1.reshape(1, d_hid), W2, npad, bm)
    acc2a, acc2b = _sc_scatter_edge(h2p, src2, dst2, tsrc, tdst, npad,
                                    d_out, rows_per_tec // 2)
    return _tc_relu_sum(acc2a, acc2b, h2p, deg0r, deg1r,
                        b2.reshape(1, d_out), n, npad, bm)
